# Initial kernel scaffold; baseline (speedup 1.0000x reference)
import jax
import jax.numpy as jnp
from jax.experimental import pallas as pl


def kernel(x_categorical, x_numerical, tables, bn_num_g, bn_num_b, W1, b1, bn1_g, bn1_b, W2, b2, bn2_g, bn2_b, W3, b3):
    raise NotImplementedError("write your pallas kernel here")


# speedup vs baseline: 2.1663x; 2.1663x over previous
import functools

import jax
import jax.numpy as jnp
from jax import lax
from jax.experimental import pallas as pl
from jax.experimental.pallas import tpu as pltpu
from jax.experimental.pallas import tpu_sc as plsc

F_FIELDS = 26
VOCAB = 100000
EMB = 32
NUM = 13
B = 4096
CAT_DIM = F_FIELDS * EMB
L1 = 512
L2 = 256
NCLS = 2
EPS = 1e-5

TOTAL_ROWS = B * F_FIELDS
CHUNK = 128

BK = 512
KTILES = B // BK


def _sc_gather(table_flat, idx3d, n_workers, chunks_per_worker):
    mesh = plsc.VectorSubcoreMesh(core_axis_name="c", subcore_axis_name="s")
    rows_per_worker = chunks_per_worker * CHUNK

    @functools.partial(
        pl.kernel,
        mesh=mesh,
        out_type=jax.ShapeDtypeStruct((TOTAL_ROWS, EMB), jnp.float32),
        scratch_types=[
            pltpu.VMEM((chunks_per_worker, CHUNK), jnp.int32),
            pltpu.VMEM((rows_per_worker, EMB), jnp.float32),
            pltpu.SemaphoreType.DMA,
        ],
        compiler_params=pltpu.CompilerParams(use_tc_tiling_on_sc=False),
    )
    def gather_kernel(table_hbm, idx_hbm, out_hbm, idx_v, rows_v, sem):
        nc = 2
        wid = lax.axis_index("s") * nc + lax.axis_index("c")
        base = wid * rows_per_worker
        pltpu.sync_copy(idx_hbm.at[wid], idx_v)

        def chunk_body(j, _):
            pltpu.async_copy(
                table_hbm.at[idx_v.at[j]],
                rows_v.at[pl.ds(j * CHUNK, CHUNK)],
                sem,
            ).wait()
            return 0

        lax.fori_loop(0, chunks_per_worker, chunk_body, 0)
        pltpu.sync_copy(rows_v, out_hbm.at[pl.ds(base, rows_per_worker)])

    return gather_kernel(table_flat, idx3d)


def _mlp_body(xc_ref, xn_ref, g0_ref, b0_ref, w1c_ref, w1n_ref, b1_ref,
              g1_ref, bb1_ref, w2_ref, b2_ref, g2_ref, bb2_ref, w3_ref,
              b3_ref, out_ref, h1_ref, h2_ref, s1_ref, s2_ref, a0_ref,
              a1_ref, a2_ref):
    p = pl.program_id(0)
    k = pl.program_id(1)
    ds = pl.ds(k * BK, BK)

    @pl.when(p == 0)
    def _phase0():
        @pl.when(k == 0)
        def _init0():
            xn = xn_ref[...]
            mu = jnp.mean(xn, axis=0, keepdims=True)
            var = jnp.mean(xn * xn, axis=0, keepdims=True) - mu * mu
            a = g0_ref[...] * lax.rsqrt(var + EPS)
            a0_ref[0:1, :] = a
            a0_ref[1:2, :] = b0_ref[...] - mu * a

        xn_t = xn_ref[ds, :] * a0_ref[0:1, :] + a0_ref[1:2, :]
        h = jnp.dot(xc_ref[...], w1c_ref[...], preferred_element_type=jnp.float32)
        h += jnp.dot(xn_t, w1n_ref[...], preferred_element_type=jnp.float32)
        h = jnp.maximum(h + b1_ref[...], 0.0)
        h1_ref[ds, :] = h
        col = jnp.sum(h, axis=0, keepdims=True)
        colsq = jnp.sum(h * h, axis=0, keepdims=True)

        @pl.when(k == 0)
        def _s1_init():
            s1_ref[0:1, :] = col
            s1_ref[1:2, :] = colsq

        @pl.when(k > 0)
        def _s1_acc():
            s1_ref[0:1, :] += col
            s1_ref[1:2, :] += colsq

    @pl.when(p == 1)
    def _phase1():
        @pl.when(k == 0)
        def _init1():
            mu = s1_ref[0:1, :] * (1.0 / B)
            var = s1_ref[1:2, :] * (1.0 / B) - mu * mu
            a = g1_ref[...] * lax.rsqrt(var + EPS)
            a1_ref[0:1, :] = a
            a1_ref[1:2, :] = bb1_ref[...] - mu * a

        ht = h1_ref[ds, :] * a1_ref[0:1, :] + a1_ref[1:2, :]
        h = jnp.dot(ht, w2_ref[...], preferred_element_type=jnp.float32)
        h = jnp.maximum(h + b2_ref[...], 0.0)
        h2_ref[ds, :] = h
        col = jnp.sum(h, axis=0, keepdims=True)
        colsq = jnp.sum(h * h, axis=0, keepdims=True)

        @pl.when(k == 0)
        def _s2_init():
            s2_ref[0:1, :] = col
            s2_ref[1:2, :] = colsq

        @pl.when(k > 0)
        def _s2_acc():
            s2_ref[0:1, :] += col
            s2_ref[1:2, :] += colsq

    @pl.when(p == 2)
    def _phase2():
        @pl.when(k == 0)
        def _init2():
            mu = s2_ref[0:1, :] * (1.0 / B)
            var = s2_ref[1:2, :] * (1.0 / B) - mu * mu
            a = g2_ref[...] * lax.rsqrt(var + EPS)
            a2_ref[0:1, :] = a
            a2_ref[1:2, :] = bb2_ref[...] - mu * a

        ht = h2_ref[ds, :] * a2_ref[0:1, :] + a2_ref[1:2, :]
        out = jnp.dot(ht, w3_ref[...], preferred_element_type=jnp.float32)
        out_ref[...] = out + b3_ref[...]


def _mlp(x_cat, x_num, bn_num_g, bn_num_b, W1c, W1n, b1, bn1_g, bn1_b,
         W2, b2, bn2_g, bn2_b, W3, b3):
    row2 = lambda v: v.reshape(1, -1)
    full = lambda shape: pl.BlockSpec(shape, lambda p, k: (0, 0))
    grid = (3, KTILES)
    return pl.pallas_call(
        _mlp_body,
        grid=grid,
        in_specs=[
            pl.BlockSpec((BK, CAT_DIM), lambda p, k: (jnp.where(p == 0, k, 0), 0)),
            full((B, NUM)),
            full((1, NUM)), full((1, NUM)),
            full((CAT_DIM, L1)), full((NUM, L1)), full((1, L1)),
            full((1, L1)), full((1, L1)),
            full((L1, L2)), full((1, L2)),
            full((1, L2)), full((1, L2)),
            full((L2, NCLS)), full((1, NCLS)),
        ],
        out_specs=pl.BlockSpec((BK, NCLS), lambda p, k: (jnp.where(p == 2, k, 0), 0)),
        out_shape=jax.ShapeDtypeStruct((B, NCLS), jnp.float32),
        scratch_shapes=[
            pltpu.VMEM((B, L1), jnp.float32),
            pltpu.VMEM((B, L2), jnp.float32),
            pltpu.VMEM((2, L1), jnp.float32),
            pltpu.VMEM((2, L2), jnp.float32),
            pltpu.VMEM((2, NUM), jnp.float32),
            pltpu.VMEM((2, L1), jnp.float32),
            pltpu.VMEM((2, L2), jnp.float32),
        ],
        compiler_params=pltpu.CompilerParams(
            dimension_semantics=("arbitrary", "arbitrary"),
        ),
    )(x_cat, x_num, row2(bn_num_g), row2(bn_num_b), W1c, W1n, row2(b1),
      row2(bn1_g), row2(bn1_b), W2, row2(b2), row2(bn2_g), row2(bn2_b),
      W3, row2(b3))


def kernel(x_categorical, x_numerical, tables, bn_num_g, bn_num_b,
           W1, b1, bn1_g, bn1_b, W2, b2, bn2_g, bn2_b, W3, b3):
    n_workers = 32
    rows_per_worker = TOTAL_ROWS // n_workers
    chunks_per_worker = rows_per_worker // CHUNK

    offs = (jnp.arange(F_FIELDS, dtype=jnp.int32) * VOCAB)[None, :]
    idx_flat = (x_categorical.astype(jnp.int32) + offs).reshape(-1)
    idx3d = idx_flat.reshape(n_workers, chunks_per_worker, CHUNK)
    table_flat = tables.reshape(F_FIELDS * VOCAB, EMB)

    rows = _sc_gather(table_flat, idx3d, n_workers, chunks_per_worker)
    x_cat = rows.reshape(B, CAT_DIM)

    W1c = W1[:CAT_DIM, :]
    W1n = W1[CAT_DIM:, :]
    return _mlp(x_cat, x_numerical, bn_num_g, bn_num_b, W1c, W1n, b1,
                bn1_g, bn1_b, W2, b2, bn2_g, bn2_b, W3, b3)
